# baseline (speedup 1.0000x reference)
import functools

import jax
import jax.numpy as jnp
from jax import lax
from jax.experimental import pallas as pl
from jax.experimental.pallas import tpu as pltpu
from jax.experimental.pallas import tpu_sc as plsc

B = 16
N = 4096
HALF = N // 2
MAX_INCLUDED = 512
SQ_CUT = 1.0
L = 16
HCHUNKS = HALF // L
OUT_WORDS = MAX_INCLUDED * 3
LSTR = HALF + L


def _sc_body(coords_hbm, ref_hbm, out_hbm, cbuf, lbuf, nbuf, obuf, cntbuf,
             ncnt, shared_buf, shared_cnt):
    c = lax.axis_index("c")
    s = lax.axis_index("s")
    t = s // 2
    h = s % 2
    row = c * (B // 2) + t

    r3 = row * 3
    pltpu.sync_copy(coords_hbm.at[r3, pl.ds(h * HALF, HALF)],
                    cbuf.at[pl.ds(0, HALF)])
    pltpu.sync_copy(coords_hbm.at[r3 + 1, pl.ds(h * HALF, HALF)],
                    cbuf.at[pl.ds(HALF, HALF)])
    pltpu.sync_copy(coords_hbm.at[r3 + 2, pl.ds(h * HALF, HALF)],
                    cbuf.at[pl.ds(2 * HALF, HALF)])
    pltpu.sync_copy(ref_hbm.at[row], cntbuf)

    lane = lax.iota(jnp.int32, L)
    czero = lane >> 4
    zeros_i = jnp.zeros((L,), jnp.int32)

    rx = cntbuf[pl.ds(0, L)]
    ry = cntbuf[pl.ds(L, L)]
    rz = cntbuf[pl.ds(2 * L, L)]

    def body(i, off):
        base = i * L
        x = cbuf[pl.ds(base, L)]
        y = cbuf[pl.ds(HALF + base, L)]
        z = cbuf[pl.ds(2 * HALF + base, L)]
        dx = x - rx
        dy = y - ry
        dz = z - rz
        d2 = dx * dx + dy * dy + dz * dz
        m = d2 <= SQ_CUT
        plsc.store_compressed(lbuf.at[pl.ds(off, L)], dx, mask=m)
        plsc.store_compressed(lbuf.at[pl.ds(LSTR + off, L)], dy, mask=m)
        plsc.store_compressed(lbuf.at[pl.ds(2 * LSTR + off, L)], dz, mask=m)
        return off + plsc.all_reduce_population_count(m)[0]

    cnt_s = lax.fori_loop(0, HCHUNKS, body, 0, unroll=8)
    cnt = zeros_i + cnt_s

    ncnt[pl.ds(0, L)] = cnt
    sb = s * OUT_WORDS
    pltpu.sync_copy(lbuf.at[pl.ds(0, MAX_INCLUDED)],
                    shared_buf.at[pl.ds(sb, MAX_INCLUDED)])
    pltpu.sync_copy(lbuf.at[pl.ds(LSTR, MAX_INCLUDED)],
                    shared_buf.at[pl.ds(sb + MAX_INCLUDED, MAX_INCLUDED)])
    pltpu.sync_copy(lbuf.at[pl.ds(2 * LSTR, MAX_INCLUDED)],
                    shared_buf.at[pl.ds(sb + 2 * MAX_INCLUDED, MAX_INCLUDED)])
    pltpu.sync_copy(ncnt, shared_cnt.at[pl.ds(s * L, L)])
    plsc.subcore_barrier()

    nb = s + 1 - 2 * h
    pltpu.sync_copy(shared_buf.at[pl.ds(nb * OUT_WORDS, OUT_WORDS)], nbuf)
    pltpu.sync_copy(shared_cnt.at[pl.ds(nb * L, L)], ncnt)
    ncv = ncnt[pl.ds(0, L)]
    c0v = jnp.where(h == 0, cnt, ncv)
    total = cnt + ncv
    joff = h * (MAX_INCLUDED // 2)

    def do_merge(buf0, str0, buf1, str1):
        def merge(j, carry):
            jloc = lane + j * L
            jvec = jloc + joff
            sel0 = jvec < c0v
            sel1 = (~sel0) & (jvec < total)
            idx1 = jnp.clip(jvec - c0v, 0, MAX_INCLUDED - 1)
            fbase = jloc * 3
            for k in range(3):
                v0 = plsc.load_gather(buf0, [jvec + k * str0])
                v1 = plsc.load_gather(buf1, [idx1 + k * str1])
                v = jnp.where(sel0, v0, jnp.where(sel1, v1, 0.0))
                plsc.store_scatter(obuf, [fbase + k], v)
            return carry
        lax.fori_loop(0, MAX_INCLUDED // (2 * L), merge, 0, unroll=4)

    @pl.when(h == 0)
    def _():
        do_merge(lbuf, LSTR, nbuf, MAX_INCLUDED)

    @pl.when(h == 1)
    def _():
        do_merge(nbuf, MAX_INCLUDED, lbuf, LSTR)

    pltpu.sync_copy(
        obuf.at[pl.ds(0, OUT_WORDS // 2)],
        out_hbm.at[row, pl.ds(h * (OUT_WORDS // 2), OUT_WORDS // 2)],
    )


@jax.jit
def _run(coords_t, ref_pad):
    mesh = plsc.VectorSubcoreMesh(core_axis_name="c", subcore_axis_name="s")
    k = functools.partial(
        pl.kernel,
        mesh=mesh,
        out_type=jax.ShapeDtypeStruct((B, OUT_WORDS), jnp.float32),
        compiler_params=pltpu.CompilerParams(
            needs_layout_passes=False,
            skip_device_barrier=True,
        ),
        scratch_types=[
            pltpu.VMEM((3 * HALF,), jnp.float32),
            pltpu.VMEM((3 * LSTR,), jnp.float32),
            pltpu.VMEM((OUT_WORDS,), jnp.float32),
            pltpu.VMEM((OUT_WORDS,), jnp.float32),
            pltpu.VMEM((3 * L,), jnp.float32),
            pltpu.VMEM((L,), jnp.int32),
            pltpu.VMEM_SHARED((16 * OUT_WORDS,), jnp.float32),
            pltpu.VMEM_SHARED((16 * L,), jnp.int32),
        ],
    )(_sc_body)
    return k(coords_t, ref_pad)


def kernel(coords, ref):
    coords_t = coords.transpose(0, 2, 1).reshape(B * 3, N)
    ref_pad = jnp.broadcast_to(ref[:, :, None], (B, 3, L)).reshape(B, 3 * L)
    out = _run(coords_t, ref_pad)
    return out.reshape(B, MAX_INCLUDED, 3)


# speedup vs baseline: 1.0078x; 1.0078x over previous
import functools

import jax
import jax.numpy as jnp
from jax import lax
from jax.experimental import pallas as pl
from jax.experimental.pallas import tpu as pltpu
from jax.experimental.pallas import tpu_sc as plsc

B = 16
N = 4096
HALF = N // 2
MAX_INCLUDED = 512
SQ_CUT = 1.0
L = 16
HCHUNKS = HALF // L
OUT_WORDS = MAX_INCLUDED * 3
LSTR = HALF + L


def _sc_body(coords_hbm, ref_hbm, out_hbm, cbuf, lbuf, nbuf, obuf, cntbuf,
             ncnt, shared_buf, shared_cnt):
    c = lax.axis_index("c")
    s = lax.axis_index("s")
    t = s // 2
    h = s % 2
    row = c * (B // 2) + t

    r3 = row * 3
    pltpu.sync_copy(coords_hbm.at[r3, pl.ds(h * HALF, HALF)],
                    cbuf.at[pl.ds(0, HALF)])
    pltpu.sync_copy(coords_hbm.at[r3 + 1, pl.ds(h * HALF, HALF)],
                    cbuf.at[pl.ds(HALF, HALF)])
    pltpu.sync_copy(coords_hbm.at[r3 + 2, pl.ds(h * HALF, HALF)],
                    cbuf.at[pl.ds(2 * HALF, HALF)])
    pltpu.sync_copy(ref_hbm.at[row], cntbuf)

    lane = lax.iota(jnp.int32, L)
    zeros_i = jnp.zeros((L,), jnp.int32)

    rx = cntbuf[pl.ds(0, L)]
    ry = cntbuf[pl.ds(L, L)]
    rz = cntbuf[pl.ds(2 * L, L)]

    def body(i, off):
        base = i * L
        x = cbuf[pl.ds(base, L)]
        y = cbuf[pl.ds(HALF + base, L)]
        z = cbuf[pl.ds(2 * HALF + base, L)]
        dx = x - rx
        dy = y - ry
        dz = z - rz
        d2 = dx * dx + dy * dy + dz * dz
        m = d2 <= SQ_CUT
        plsc.store_compressed(lbuf.at[pl.ds(off, L)], dx, mask=m)
        plsc.store_compressed(lbuf.at[pl.ds(LSTR + off, L)], dy, mask=m)
        plsc.store_compressed(lbuf.at[pl.ds(2 * LSTR + off, L)], dz, mask=m)
        return off + plsc.all_reduce_population_count(m)[0]

    cnt_s = lax.fori_loop(0, HCHUNKS, body, 0, unroll=8)
    cnt = zeros_i + cnt_s

    ncnt[pl.ds(0, L)] = cnt
    sb = s * OUT_WORDS
    pltpu.sync_copy(lbuf.at[pl.ds(0, MAX_INCLUDED)],
                    shared_buf.at[pl.ds(sb, MAX_INCLUDED)])
    pltpu.sync_copy(lbuf.at[pl.ds(LSTR, MAX_INCLUDED)],
                    shared_buf.at[pl.ds(sb + MAX_INCLUDED, MAX_INCLUDED)])
    pltpu.sync_copy(lbuf.at[pl.ds(2 * LSTR, MAX_INCLUDED)],
                    shared_buf.at[pl.ds(sb + 2 * MAX_INCLUDED, MAX_INCLUDED)])
    pltpu.sync_copy(ncnt, shared_cnt.at[pl.ds(s * L, L)])
    plsc.subcore_barrier()

    @pl.when(h == 0)
    def _():
        pltpu.sync_copy(shared_buf.at[pl.ds((s + 1) * OUT_WORDS, OUT_WORDS)], nbuf)
        pltpu.sync_copy(shared_cnt.at[pl.ds((s + 1) * L, L)], ncnt)
        c1 = ncnt[pl.ds(0, L)]
        total = cnt + c1

        def merge(j, carry):
            jvec = lane + j * L
            sel0 = jvec < cnt
            sel1 = (~sel0) & (jvec < total)
            idx1 = jnp.clip(jvec - cnt, 0, MAX_INCLUDED - 1)
            fbase = jvec * 3
            for k in range(3):
                v0 = plsc.load_gather(lbuf, [jvec + k * LSTR])
                v1 = plsc.load_gather(nbuf, [idx1 + k * MAX_INCLUDED])
                v = jnp.where(sel0, v0, jnp.where(sel1, v1, 0.0))
                plsc.store_scatter(obuf, [fbase + k], v)
            return carry

        lax.fori_loop(0, MAX_INCLUDED // L, merge, 0, unroll=4)
        pltpu.sync_copy(obuf, out_hbm.at[row])


@jax.jit
def _run(coords_t, ref_pad):
    mesh = plsc.VectorSubcoreMesh(core_axis_name="c", subcore_axis_name="s")
    k = functools.partial(
        pl.kernel,
        mesh=mesh,
        out_type=jax.ShapeDtypeStruct((B, OUT_WORDS), jnp.float32),
        compiler_params=pltpu.CompilerParams(
            needs_layout_passes=False,
            skip_device_barrier=True,
        ),
        scratch_types=[
            pltpu.VMEM((3 * HALF,), jnp.float32),
            pltpu.VMEM((3 * LSTR,), jnp.float32),
            pltpu.VMEM((OUT_WORDS,), jnp.float32),
            pltpu.VMEM((OUT_WORDS,), jnp.float32),
            pltpu.VMEM((3 * L,), jnp.float32),
            pltpu.VMEM((L,), jnp.int32),
            pltpu.VMEM_SHARED((16 * OUT_WORDS,), jnp.float32),
            pltpu.VMEM_SHARED((16 * L,), jnp.int32),
        ],
    )(_sc_body)
    return k(coords_t, ref_pad)


def kernel(coords, ref):
    coords_t = coords.transpose(0, 2, 1).reshape(B * 3, N)
    ref_pad = jnp.broadcast_to(ref[:, :, None], (B, 3, L)).reshape(B, 3 * L)
    out = _run(coords_t, ref_pad)
    return out.reshape(B, MAX_INCLUDED, 3)
